# baseline (speedup 1.0000x reference)
import dataclasses
import functools

import jax
import jax.numpy as jnp
from jax import lax
from jax.experimental import pallas as pl
from jax.experimental.pallas import tpu as pltpu
from jax.experimental.pallas import tpu_sc as plsc

N = 10000
E = 320000
DF = 128
DE = 16
HE = 16
HN = 128

NT = 10240
NCHUNK = NT // 128
ECHUNK = E // 128

NSHARD = 4
KF = 2
EPS = E // NSHARD
CPS = ECHUNK // NSHARD
EB = 3200
CB = EB // 128
NBLK = EPS // EB


def _xsr_body(x_ref, w_ref, o_ref):
    r = lax.dot_general(
        w_ref[...], x_ref[...], (((1,), (1,)), ((), ())),
        preferred_element_type=jnp.float32)
    for c in range(8):
        o_ref[:, c, :] = r[:, 128 * c:128 * (c + 1)]


def _node_proj_t(x_pad, w_catT):
    return pl.pallas_call(
        _xsr_body,
        grid=(NCHUNK // 8,),
        in_specs=[
            pl.BlockSpec((1024, DF), lambda i: (i, 0)),
            pl.BlockSpec((2 * HE, DF), lambda i: (0, 0)),
        ],
        out_specs=pl.BlockSpec((2 * HE, 8, 128), lambda i: (0, i, 0)),
        out_shape=jax.ShapeDtypeStruct((2 * HE, NCHUNK, 128), jnp.float32),
    )(x_pad, w_catT)


def _eproj_body(ea_ref, w_ref, b_ref, o_ref):
    r = jnp.dot(w_ref[...], ea_ref[...],
                preferred_element_type=jnp.float32) + b_ref[...]
    for c in range(10):
        o_ref[c, :, :] = r[:, 128 * c:128 * (c + 1)]


def _edge_proj_t(eaT, w_eeT, b_e):
    blk = 1280
    return pl.pallas_call(
        _eproj_body,
        grid=(E // blk,),
        in_specs=[
            pl.BlockSpec((DE, blk), lambda i: (0, i)),
            pl.BlockSpec((HE, DE), lambda i: (0, 0)),
            pl.BlockSpec((HE, 1), lambda i: (0, 0)),
        ],
        out_specs=pl.BlockSpec((blk // 128, HE, 128), lambda i: (i, 0, 0)),
        out_shape=jax.ShapeDtypeStruct((ECHUNK, HE, 128), jnp.float32),
    )(eaT, w_eeT, b_e)


def _nodeup_body(x_ref, a_ref, wx_ref, wa_ref, b_ref, o_ref):
    acc = jnp.dot(x_ref[...], wx_ref[...], preferred_element_type=jnp.float32)
    a = a_ref[:, 0] + a_ref[:, 1] + a_ref[:, 2] + a_ref[:, 3]
    for c in range(8):
        ac = lax.dot_general(
            a[:, c, :], wa_ref[...], (((0,), (0,)), ((), ())),
            preferred_element_type=jnp.float32)
        o_ref[128 * c:128 * (c + 1), :] = jnp.maximum(
            acc[128 * c:128 * (c + 1), :] + ac + b_ref[...], 0.0)


def _node_update(x_pad, aggs, w_x, w_a, b_n):
    blk = 1024
    return pl.pallas_call(
        _nodeup_body,
        grid=(NT // blk,),
        in_specs=[
            pl.BlockSpec((blk, DF), lambda i: (i, 0)),
            pl.BlockSpec((HE, NSHARD, blk // 128, 128), lambda i: (0, 0, i, 0)),
            pl.BlockSpec((DF, HN), lambda i: (0, 0)),
            pl.BlockSpec((HE, HN), lambda i: (0, 0)),
            pl.BlockSpec((1, HN), lambda i: (0, 0)),
        ],
        out_specs=pl.BlockSpec((blk, HN), lambda i: (i, 0)),
        out_shape=jax.ShapeDtypeStruct((NT, HN), jnp.float32),
    )(x_pad, aggs, w_x, w_a, b_n)


def _sc_compiler_params():
    cp = pltpu.CompilerParams(use_tc_tiling_on_sc=False)
    if "needs_layout_passes" in pltpu.CompilerParams.__dataclass_fields__:
        cp = dataclasses.replace(cp, needs_layout_passes=False)
    return cp


def _sc_edge_agg(src, dst, xsrT, epT, zeros):
    mesh = plsc.VectorSubcoreMesh(core_axis_name="c", subcore_axis_name="s")

    @functools.partial(
        pl.kernel,
        out_type=jax.ShapeDtypeStruct((HE, NSHARD, NCHUNK, 128), jnp.float32),
        mesh=mesh,
        compiler_params=_sc_compiler_params(),
        scratch_types=[
            pltpu.VMEM((NCHUNK, 128), jnp.float32),
            pltpu.VMEM((NCHUNK, 128), jnp.float32),
            pltpu.VMEM((NCHUNK, 128), jnp.float32),
            pltpu.VMEM((NCHUNK, 128), jnp.float32),
            pltpu.VMEM((NCHUNK, 128), jnp.float32),
            pltpu.VMEM((NCHUNK, 128), jnp.float32),
            pltpu.VMEM((EB,), jnp.int32),
            pltpu.VMEM((EB,), jnp.int32),
            pltpu.VMEM((CB, KF, 128), jnp.float32),
            pltpu.VMEM((EB,), jnp.int32),
            pltpu.VMEM((EB,), jnp.int32),
            pltpu.VMEM((CB, KF, 128), jnp.float32),
            pltpu.SemaphoreType.DMA,
            pltpu.SemaphoreType.DMA,
            pltpu.SemaphoreType.DMA,
        ],
    )
    def sc_kernel(src_hbm, dst_hbm, xsr_hbm, ep_hbm, z_hbm, out_hbm,
                  txs0, txs1, txr0, txr1, agg0, agg1,
                  srcA, dstA, epA, srcB, dstB, epB,
                  semT, semA, semB):
        cid = lax.axis_index("c")
        sid = lax.axis_index("s")
        wid = sid * 2 + cid
        g = wid % (HE // KF)
        h = wid // (HE // KF)
        f0 = g * KF

        tbl_xs = [txs0, txs1]
        tbl_xr = [txr0, txr1]
        agg = [agg0, agg1]

        for k in range(KF):
            pltpu.async_copy(xsr_hbm.at[f0 + k], tbl_xs[k], semT)
            pltpu.async_copy(xsr_hbm.at[HE + f0 + k], tbl_xr[k], semT)
            pltpu.async_copy(z_hbm, agg[k], semT)
        for k in range(KF):
            pltpu.make_async_copy(xsr_hbm.at[f0 + k], tbl_xs[k], semT).wait()
            pltpu.make_async_copy(xsr_hbm.at[HE + f0 + k], tbl_xr[k],
                                  semT).wait()
            pltpu.make_async_copy(z_hbm, agg[k], semT).wait()

        def issue(t, sv, dv, ev, sem):
            e0 = h * EPS + t * EB
            c0 = h * CPS + t * CB
            pltpu.async_copy(src_hbm.at[pl.ds(e0, EB)], sv, sem)
            pltpu.async_copy(dst_hbm.at[pl.ds(e0, EB)], dv, sem)
            pltpu.async_copy(ep_hbm.at[pl.ds(c0, CB), pl.ds(f0, KF)],
                             ev, sem)

        def drain(t, sv, dv, ev, sem):
            e0 = h * EPS + t * EB
            c0 = h * CPS + t * CB
            pltpu.make_async_copy(src_hbm.at[pl.ds(e0, EB)], sv, sem).wait()
            pltpu.make_async_copy(dst_hbm.at[pl.ds(e0, EB)], dv, sem).wait()
            pltpu.make_async_copy(ep_hbm.at[pl.ds(c0, CB), pl.ds(f0, KF)],
                                  ev, sem).wait()

        def compute(sv, dv, ev):
            @pl.loop(0, CB)
            def _(r):
                for gg in range(8):
                    off = r * 128 + gg * 16
                    s = sv[pl.ds(off, 16)]
                    d = dv[pl.ds(off, 16)]
                    rs = lax.shift_right_logical(s, 7)
                    cs = lax.bitwise_and(s, 127)
                    rd = lax.shift_right_logical(d, 7)
                    cd = lax.bitwise_and(d, 127)
                    for k in range(KF):
                        e = ev[r, k, pl.ds(gg * 16, 16)]
                        gs = plsc.load_gather(tbl_xs[k], [rs, cs])
                        gr = plsc.load_gather(tbl_xr[k], [rd, cd])
                        v = jnp.maximum(e + gs + gr, 0.0)
                        plsc.addupdate_scatter(agg[k], [rd, cd], v)

        issue(0, srcA, dstA, epA, semA)

        @pl.loop(0, (NBLK - 1) // 2)
        def _(t):
            t0 = t * 2
            drain(t0, srcA, dstA, epA, semA)
            issue(t0 + 1, srcB, dstB, epB, semB)
            compute(srcA, dstA, epA)
            drain(t0 + 1, srcB, dstB, epB, semB)
            issue(t0 + 2, srcA, dstA, epA, semA)
            compute(srcB, dstB, epB)

        drain(NBLK - 1, srcA, dstA, epA, semA)
        compute(srcA, dstA, epA)

        for k in range(KF):
            pltpu.sync_copy(agg[k], out_hbm.at[f0 + k, h])

    return sc_kernel(src, dst, xsrT, epT, zeros)


def kernel(x, edge_attr, edge_index, W_e, b_e, W_n, b_n):
    src = edge_index[0]
    dst = edge_index[1]

    w_ee = W_e[:DE]
    w_s = W_e[DE:DE + DF]
    w_r = W_e[DE + DF:]
    w_catT = jnp.concatenate([w_s, w_r], axis=1).T

    x_pad = jnp.zeros((NT, DF), jnp.float32).at[:N].set(x)
    xsrT = _node_proj_t(x_pad, w_catT)

    eaT = edge_attr.T
    epT = _edge_proj_t(eaT, w_ee.T, b_e.reshape(HE, 1))

    zeros = jnp.zeros((NCHUNK, 128), jnp.float32)
    aggs = _sc_edge_agg(src, dst, xsrT, epT, zeros)

    w_x = W_n[:DF]
    w_a = W_n[DF:]
    out = _node_update(x_pad, aggs, w_x, w_a, b_n.reshape(1, HN))
    return out[:N]


# speedup vs baseline: 2.3358x; 2.3358x over previous
import dataclasses
import functools

import jax
import jax.numpy as jnp
from jax import lax
from jax.experimental import pallas as pl
from jax.experimental.pallas import tpu as pltpu
from jax.experimental.pallas import tpu_sc as plsc

N = 10000
E = 320000
DF = 128
DE = 16
HE = 16
HN = 128

NT = 10240
NCHUNK = NT // 128
ECHUNK = E // 128

NSHARD = 4
KF = 2
EPS = E // NSHARD
CPS = ECHUNK // NSHARD
EB = 3200
CB = EB // 128
NBLK = EPS // EB


def _xsr_body(x_ref, w_ref, o_ref):
    r = lax.dot_general(
        w_ref[...], x_ref[...], (((1,), (1,)), ((), ())),
        preferred_element_type=jnp.float32)
    for c in range(8):
        o_ref[:, c, :] = r[:, 128 * c:128 * (c + 1)]


def _node_proj_t(x_pad, w_catT):
    return pl.pallas_call(
        _xsr_body,
        grid=(NCHUNK // 8,),
        in_specs=[
            pl.BlockSpec((1024, DF), lambda i: (i, 0)),
            pl.BlockSpec((2 * HE, DF), lambda i: (0, 0)),
        ],
        out_specs=pl.BlockSpec((2 * HE, 8, 128), lambda i: (0, i, 0)),
        out_shape=jax.ShapeDtypeStruct((2 * HE, NCHUNK, 128), jnp.float32),
    )(x_pad, w_catT)


def _eproj_body(ea_ref, w_ref, b_ref, o_ref):
    r = jnp.dot(w_ref[...], ea_ref[...],
                preferred_element_type=jnp.float32) + b_ref[...]
    for c in range(50):
        o_ref[c, :, :] = r[:, 128 * c:128 * (c + 1)]


def _edge_proj_t(eaT, w_eeT, b_e):
    blk = 6400
    return pl.pallas_call(
        _eproj_body,
        grid=(E // blk,),
        in_specs=[
            pl.BlockSpec((DE, blk), lambda i: (0, i)),
            pl.BlockSpec((HE, DE), lambda i: (0, 0)),
            pl.BlockSpec((HE, 1), lambda i: (0, 0)),
        ],
        out_specs=pl.BlockSpec((blk // 128, HE, 128), lambda i: (i, 0, 0)),
        out_shape=jax.ShapeDtypeStruct((ECHUNK, HE, 128), jnp.float32),
    )(eaT, w_eeT, b_e)


def _nodeup_body(x_ref, a_ref, wx_ref, wa_ref, b_ref, o_ref):
    acc = jnp.dot(x_ref[...], wx_ref[...], preferred_element_type=jnp.float32)
    a = a_ref[:, 0] + a_ref[:, 1] + a_ref[:, 2] + a_ref[:, 3]
    for c in range(8):
        ac = lax.dot_general(
            a[:, c, :], wa_ref[...], (((0,), (0,)), ((), ())),
            preferred_element_type=jnp.float32)
        o_ref[128 * c:128 * (c + 1), :] = jnp.maximum(
            acc[128 * c:128 * (c + 1), :] + ac + b_ref[...], 0.0)


def _node_update(x_pad, aggs, w_x, w_a, b_n):
    blk = 1024
    return pl.pallas_call(
        _nodeup_body,
        grid=(NT // blk,),
        in_specs=[
            pl.BlockSpec((blk, DF), lambda i: (i, 0)),
            pl.BlockSpec((HE, NSHARD, blk // 128, 128), lambda i: (0, 0, i, 0)),
            pl.BlockSpec((DF, HN), lambda i: (0, 0)),
            pl.BlockSpec((HE, HN), lambda i: (0, 0)),
            pl.BlockSpec((1, HN), lambda i: (0, 0)),
        ],
        out_specs=pl.BlockSpec((blk, HN), lambda i: (i, 0)),
        out_shape=jax.ShapeDtypeStruct((NT, HN), jnp.float32),
    )(x_pad, aggs, w_x, w_a, b_n)


def _sc_compiler_params():
    cp = pltpu.CompilerParams(use_tc_tiling_on_sc=False)
    if "needs_layout_passes" in pltpu.CompilerParams.__dataclass_fields__:
        cp = dataclasses.replace(cp, needs_layout_passes=False)
    return cp


def _sc_edge_agg(src, dst, xsrT, epT, zeros):
    mesh = plsc.VectorSubcoreMesh(core_axis_name="c", subcore_axis_name="s")

    @functools.partial(
        pl.kernel,
        out_type=jax.ShapeDtypeStruct((HE, NSHARD, NCHUNK, 128), jnp.float32),
        mesh=mesh,
        compiler_params=_sc_compiler_params(),
        scratch_types=[
            pltpu.VMEM((NCHUNK, 128), jnp.float32),
            pltpu.VMEM((NCHUNK, 128), jnp.float32),
            pltpu.VMEM((NCHUNK, 128), jnp.float32),
            pltpu.VMEM((NCHUNK, 128), jnp.float32),
            pltpu.VMEM((NCHUNK, 128), jnp.float32),
            pltpu.VMEM((NCHUNK, 128), jnp.float32),
            pltpu.VMEM((EB,), jnp.int32),
            pltpu.VMEM((EB,), jnp.int32),
            pltpu.VMEM((CB, KF, 128), jnp.float32),
            pltpu.VMEM((EB,), jnp.int32),
            pltpu.VMEM((EB,), jnp.int32),
            pltpu.VMEM((CB, KF, 128), jnp.float32),
            pltpu.SemaphoreType.DMA,
            pltpu.SemaphoreType.DMA,
            pltpu.SemaphoreType.DMA,
        ],
    )
    def sc_kernel(src_hbm, dst_hbm, xsr_hbm, ep_hbm, z_hbm, out_hbm,
                  txs0, txs1, txr0, txr1, agg0, agg1,
                  srcA, dstA, epA, srcB, dstB, epB,
                  semT, semA, semB):
        cid = lax.axis_index("c")
        sid = lax.axis_index("s")
        wid = sid * 2 + cid
        g = wid % (HE // KF)
        h = wid // (HE // KF)
        f0 = g * KF

        tbl_xs = [txs0, txs1]
        tbl_xr = [txr0, txr1]
        agg = [agg0, agg1]

        for k in range(KF):
            pltpu.async_copy(xsr_hbm.at[f0 + k], tbl_xs[k], semT)
            pltpu.async_copy(xsr_hbm.at[HE + f0 + k], tbl_xr[k], semT)
            pltpu.async_copy(z_hbm, agg[k], semT)
        for k in range(KF):
            pltpu.make_async_copy(xsr_hbm.at[f0 + k], tbl_xs[k], semT).wait()
            pltpu.make_async_copy(xsr_hbm.at[HE + f0 + k], tbl_xr[k],
                                  semT).wait()
            pltpu.make_async_copy(z_hbm, agg[k], semT).wait()

        def issue(t, sv, dv, ev, sem):
            e0 = h * EPS + t * EB
            c0 = h * CPS + t * CB
            pltpu.async_copy(src_hbm.at[pl.ds(e0, EB)], sv, sem)
            pltpu.async_copy(dst_hbm.at[pl.ds(e0, EB)], dv, sem)
            pltpu.async_copy(ep_hbm.at[pl.ds(c0, CB), pl.ds(f0, KF)],
                             ev, sem)

        def drain(t, sv, dv, ev, sem):
            e0 = h * EPS + t * EB
            c0 = h * CPS + t * CB
            pltpu.make_async_copy(src_hbm.at[pl.ds(e0, EB)], sv, sem).wait()
            pltpu.make_async_copy(dst_hbm.at[pl.ds(e0, EB)], dv, sem).wait()
            pltpu.make_async_copy(ep_hbm.at[pl.ds(c0, CB), pl.ds(f0, KF)],
                                  ev, sem).wait()

        def compute(sv, dv, ev):
            @functools.partial(plsc.parallel_loop, 0, CB, unroll=2)
            def _(r):
                for gg in range(8):
                    off = r * 128 + gg * 16
                    s = sv[pl.ds(off, 16)]
                    d = dv[pl.ds(off, 16)]
                    rs = lax.shift_right_logical(s, 7)
                    cs = lax.bitwise_and(s, 127)
                    rd = lax.shift_right_logical(d, 7)
                    cd = lax.bitwise_and(d, 127)
                    for k in range(KF):
                        e = ev[r, k, pl.ds(gg * 16, 16)]
                        gs = plsc.load_gather(tbl_xs[k], [rs, cs])
                        gr = plsc.load_gather(tbl_xr[k], [rd, cd])
                        v = jnp.maximum(e + gs + gr, 0.0)
                        plsc.addupdate_scatter(agg[k], [rd, cd], v)

        issue(0, srcA, dstA, epA, semA)

        @pl.loop(0, (NBLK - 1) // 2)
        def _(t):
            t0 = t * 2
            drain(t0, srcA, dstA, epA, semA)
            issue(t0 + 1, srcB, dstB, epB, semB)
            compute(srcA, dstA, epA)
            drain(t0 + 1, srcB, dstB, epB, semB)
            issue(t0 + 2, srcA, dstA, epA, semA)
            compute(srcB, dstB, epB)

        drain(NBLK - 1, srcA, dstA, epA, semA)
        compute(srcA, dstA, epA)

        for k in range(KF):
            pltpu.sync_copy(agg[k], out_hbm.at[f0 + k, h])

    return sc_kernel(src, dst, xsrT, epT, zeros)


def kernel(x, edge_attr, edge_index, W_e, b_e, W_n, b_n):
    src = edge_index[0]
    dst = edge_index[1]

    w_ee = W_e[:DE]
    w_s = W_e[DE:DE + DF]
    w_r = W_e[DE + DF:]
    w_catT = jnp.concatenate([w_s, w_r], axis=1).T

    x_pad = jnp.zeros((NT, DF), jnp.float32).at[:N].set(x)
    xsrT = _node_proj_t(x_pad, w_catT)

    eaT = edge_attr.T
    epT = _edge_proj_t(eaT, w_ee.T, b_e.reshape(HE, 1))

    zeros = jnp.zeros((NCHUNK, 128), jnp.float32)
    aggs = _sc_edge_agg(src, dst, xsrT, epT, zeros)

    w_x = W_n[:DF]
    w_a = W_n[DF:]
    out = _node_update(x_pad, aggs, w_x, w_a, b_n.reshape(1, HN))
    return out[:N]
